# baseline (speedup 1.0000x reference)
import jax
import jax.numpy as jnp
from jax.experimental import pallas as pl
from jax.experimental.pallas import tpu as pltpu

T_BLK = 32
O_BLK = 128


def _fused_kernel(x_ref, mu_ref, rho_ref, gate_ref, bias_ref,
                  y_ref, sigma_ref, mw_ref, proto_ref, thr_ref):
    t = pl.program_id(1)

    @pl.when(t == 0)
    def _init():
        proto_ref[...] = jax.nn.softplus(rho_ref[...])
        thr_ref[...] = jnp.mean(gate_ref[...], axis=1, keepdims=True)

    proto = proto_ref[...]
    mu = mu_ref[...]
    x = x_ref[...]

    scores_t = jax.lax.dot_general(
        proto, x, (((1,), (1,)), ((), ())),
        preferred_element_type=jnp.float32)
    thr = thr_ref[...]
    soft = scores_t - thr
    hard = (scores_t > thr).astype(jnp.float32)
    mask_t = (hard + soft) - soft

    sigma_ref[...] = scores_t.T
    ymat = jax.lax.dot_general(
        x, mu, (((1,), (1,)), ((), ())),
        preferred_element_type=jnp.float32)
    y_ref[...] = mask_t.T * ymat + bias_ref[...]

    for i in range(T_BLK):
        mw_ref[i] = mask_t[:, i:i + 1] * mu


def kernel(x, mu_weight, rho_weight, gate_weight, mu_bias):
    b, s, d = x.shape
    o = mu_weight.shape[0]
    bs = b * s
    x2 = x.reshape(bs, d)
    bias2 = mu_bias.reshape(1, o)
    grid = (o // O_BLK, bs // T_BLK)

    y2, sigma, mw = pl.pallas_call(
        _fused_kernel,
        grid=grid,
        in_specs=[
            pl.BlockSpec((T_BLK, d), lambda oi, ti: (ti, 0)),
            pl.BlockSpec((O_BLK, d), lambda oi, ti: (oi, 0)),
            pl.BlockSpec((O_BLK, d), lambda oi, ti: (oi, 0)),
            pl.BlockSpec((O_BLK, d), lambda oi, ti: (oi, 0)),
            pl.BlockSpec((1, O_BLK), lambda oi, ti: (0, oi)),
        ],
        out_specs=[
            pl.BlockSpec((T_BLK, O_BLK), lambda oi, ti: (ti, oi)),
            pl.BlockSpec((T_BLK, O_BLK), lambda oi, ti: (ti, oi)),
            pl.BlockSpec((T_BLK, O_BLK, d), lambda oi, ti: (ti, oi, 0)),
        ],
        out_shape=[
            jax.ShapeDtypeStruct((bs, o), jnp.float32),
            jax.ShapeDtypeStruct((bs, o), jnp.float32),
            jax.ShapeDtypeStruct((bs, o, d), jnp.float32),
        ],
        scratch_shapes=[
            pltpu.VMEM((O_BLK, d), jnp.float32),
            pltpu.VMEM((O_BLK, 1), jnp.float32),
        ],
        compiler_params=pltpu.CompilerParams(
            dimension_semantics=("parallel", "arbitrary"),
            vmem_limit_bytes=64 * 1024 * 1024,
        ),
    )(x2, mu_weight, rho_weight, gate_weight, bias2)
    return y2.reshape(b, s, o), (sigma, mw)


# speedup vs baseline: 1.0251x; 1.0251x over previous
import jax
import jax.numpy as jnp
from jax.experimental import pallas as pl
from jax.experimental.pallas import tpu as pltpu

T_BLK = 16
O_BLK = 256


def _fused_kernel(x_ref, mu_ref, rho_ref, gate_ref, bias_ref,
                  y_ref, sigma_ref, mw_ref, proto_ref, thr_ref):
    t = pl.program_id(1)

    @pl.when(t == 0)
    def _init():
        proto_ref[...] = jax.nn.softplus(rho_ref[...])
        thr_ref[...] = jnp.mean(gate_ref[...], axis=1, keepdims=True)

    proto = proto_ref[...]
    mu = mu_ref[...]
    x = x_ref[...]

    scores_t = jax.lax.dot_general(
        proto, x, (((1,), (1,)), ((), ())),
        preferred_element_type=jnp.float32)
    thr = thr_ref[...]
    soft = scores_t - thr
    hard = (scores_t > thr).astype(jnp.float32)
    mask_t = (hard + soft) - soft

    sigma_ref[...] = scores_t.T
    ymat = jax.lax.dot_general(
        x, mu, (((1,), (1,)), ((), ())),
        preferred_element_type=jnp.float32)
    y_ref[...] = mask_t.T * ymat + bias_ref[...]

    for i in range(T_BLK):
        mw_ref[i] = mask_t[:, i:i + 1] * mu


def kernel(x, mu_weight, rho_weight, gate_weight, mu_bias):
    b, s, d = x.shape
    o = mu_weight.shape[0]
    bs = b * s
    x2 = x.reshape(bs, d)
    bias2 = mu_bias.reshape(1, o)
    grid = (o // O_BLK, bs // T_BLK)

    y2, sigma, mw = pl.pallas_call(
        _fused_kernel,
        grid=grid,
        in_specs=[
            pl.BlockSpec((T_BLK, d), lambda oi, ti: (ti, 0)),
            pl.BlockSpec((O_BLK, d), lambda oi, ti: (oi, 0)),
            pl.BlockSpec((O_BLK, d), lambda oi, ti: (oi, 0)),
            pl.BlockSpec((O_BLK, d), lambda oi, ti: (oi, 0)),
            pl.BlockSpec((1, O_BLK), lambda oi, ti: (0, oi)),
        ],
        out_specs=[
            pl.BlockSpec((T_BLK, O_BLK), lambda oi, ti: (ti, oi)),
            pl.BlockSpec((T_BLK, O_BLK), lambda oi, ti: (ti, oi)),
            pl.BlockSpec((T_BLK, O_BLK, d), lambda oi, ti: (ti, oi, 0)),
        ],
        out_shape=[
            jax.ShapeDtypeStruct((bs, o), jnp.float32),
            jax.ShapeDtypeStruct((bs, o), jnp.float32),
            jax.ShapeDtypeStruct((bs, o, d), jnp.float32),
        ],
        scratch_shapes=[
            pltpu.VMEM((O_BLK, d), jnp.float32),
            pltpu.VMEM((O_BLK, 1), jnp.float32),
        ],
        compiler_params=pltpu.CompilerParams(
            dimension_semantics=("parallel", "arbitrary"),
            vmem_limit_bytes=64 * 1024 * 1024,
        ),
    )(x2, mu_weight, rho_weight, gate_weight, bias2)
    return y2.reshape(b, s, o), (sigma, mw)
